# Initial kernel scaffold; baseline (speedup 1.0000x reference)
import jax
import jax.numpy as jnp
from jax.experimental import pallas as pl


def kernel(rel_sr_weight, rel_tg_weight, pos_sr, relation_sr, conf_sr, imp_sr, pca_sr, pos_tg, relation_tg, conf_tg, imp_tg, pca_tg):
    raise NotImplementedError("write your pallas kernel here")


# speedup vs baseline: 177.5323x; 177.5323x over previous
import functools

import jax
import jax.numpy as jnp
from jax import lax
from jax.experimental import pallas as pl
from jax.experimental.pallas import tpu as pltpu
from jax.experimental.pallas import tpu_sc as plsc

_E = 3_200_000
_N = 100_000
_RSR_PAD = 1024
_RTG_PAD = 1280

_NC = 2
_NS = 16
_NW = _NC * _NS
_L = 16

_NPAD = 100_352
_SLICE = _NPAD // _NS
_EPW = _E // _NW
_C = 4000
_NCHUNK = _EPW // _C
_IW = 80
_IR = _C // _IW


def _relw_body(a_ref, b_ref, wsr_ref, wtg_ref):
    a = a_ref[...]
    b = b_ref[...]
    pa = a / (jnp.sqrt(jnp.sum(a * a, axis=1, keepdims=True)) + 1e-8)
    pb = b / (jnp.sqrt(jnp.sum(b * b, axis=1, keepdims=True)) + 1e-8)
    sim = lax.dot_general(pa, pb, (((1,), (1,)), ((), ())),
                          preferred_element_type=jnp.float32)
    ii = lax.broadcasted_iota(jnp.int32, sim.shape, 0)
    jj = lax.broadcasted_iota(jnp.int32, sim.shape, 1)
    neg = jnp.float32(-3.0e38)
    wsr_ref[...] = jnp.max(jnp.where(jj < 1200, sim, neg), axis=1, keepdims=True)
    wtg_ref[...] = jnp.max(jnp.where(ii < 1200, sim, neg), axis=0, keepdims=True)


def _relation_w(a, b):
    a = jnp.pad(a, ((0, 1280 - a.shape[0]), (0, 0)))
    b = jnp.pad(b, ((0, 1280 - b.shape[0]), (0, 0)))
    wsr2, wtg2 = pl.pallas_call(
        _relw_body,
        out_shape=(jax.ShapeDtypeStruct((1280, 1), jnp.float32),
                   jax.ShapeDtypeStruct((1, 1280), jnp.float32)),
    )(a, b)
    return wsr2[:_RSR_PAD, 0], wtg2[0, :]


def _dis_body(dsr_ref, dtg_ref, dis_sr_ref, d2_sr_ref, dis_tg_ref, d2_tg_ref):
    for dref, oref, o2ref in ((dsr_ref, dis_sr_ref, d2_sr_ref),
                              (dtg_ref, dis_tg_ref, d2_tg_ref)):
        d = dref[0] + dref[1] + 1.0
        y = jnp.where(d > 0.0, lax.rsqrt(jnp.maximum(d, 1e-12)), 0.0)
        oref[...] = y
        o2ref[...] = y * y


def _deg_to_dis(deg_sr, deg_tg):
    shp = jax.ShapeDtypeStruct((_NPAD // 128, 128), jnp.float32)
    return pl.pallas_call(
        _dis_body,
        out_shape=(shp, shp, shp, shp),
    )(deg_sr.reshape(2, _NPAD // 128, 128), deg_tg.reshape(2, _NPAD // 128, 128))


def _make_pass1(rpad):
    mesh = plsc.VectorSubcoreMesh(core_axis_name="c", subcore_axis_name="s")

    @functools.partial(
        pl.kernel,
        mesh=mesh,
        compiler_params=pltpu.CompilerParams(needs_layout_passes=False,
                                             use_tc_tiling_on_sc=False),
        out_type=[jax.ShapeDtypeStruct((_E,), jnp.float32),
                  jax.ShapeDtypeStruct((2 * _NPAD,), jnp.float32)],
        scratch_types=[
            pltpu.VMEM((rpad,), jnp.float32),
            pltpu.VMEM((_C,), jnp.int32),
            pltpu.VMEM((_C,), jnp.float32),
            pltpu.VMEM((_C,), jnp.float32),
            pltpu.VMEM((_C,), jnp.float32),
            pltpu.VMEM((_IR, _IW), jnp.int32),
            pltpu.VMEM((_C,), jnp.float32),
            pltpu.VMEM_SHARED((_NPAD,), jnp.float32),
        ],
    )
    def pass1(w_hbm, rel_hbm, conf_hbm, imp_hbm, pca_hbm, rows2_hbm, zeros_hbm,
              vals_hbm, deg_hbm,
              w_v, rel_v, conf_v, imp_v, pca_v, rows_v, vals_v, deg_sh):
        cid = lax.axis_index("c")
        sid = lax.axis_index("s")
        wid = sid * _NC + cid
        pltpu.sync_copy(zeros_hbm.at[pl.ds(sid * _SLICE, _SLICE)],
                        deg_sh.at[pl.ds(sid * _SLICE, _SLICE)])
        pltpu.sync_copy(w_hbm, w_v)
        plsc.subcore_barrier()

        def chunk(c, carry):
            base = wid * _EPW + c * _C
            rbase = wid * (_EPW // _IW) + c * _IR
            pltpu.sync_copy(rel_hbm.at[pl.ds(base, _C)], rel_v)
            pltpu.sync_copy(conf_hbm.at[pl.ds(base, _C)], conf_v)
            pltpu.sync_copy(imp_hbm.at[pl.ds(base, _C)], imp_v)
            pltpu.sync_copy(pca_hbm.at[pl.ds(base, _C)], pca_v)
            pltpu.sync_copy(rows2_hbm.at[pl.ds(rbase, _IR)], rows_v)

            def vec(j, carry2):
                o = j * _L
                att = plsc.load_gather(w_v, [rel_v[pl.ds(o, _L)]])
                v = (conf_v[pl.ds(o, _L)] * imp_v[pl.ds(o, _L)]
                     * (0.5 * pca_v[pl.ds(o, _L)] + 0.5 * att))
                vals_v[pl.ds(o, _L)] = v
                return carry2

            lax.fori_loop(0, _C // _L, vec, 0)
            pltpu.sync_copy(vals_v, vals_hbm.at[pl.ds(base, _C)])
            for j in range(_IR):
                pltpu.sync_copy(vals_v.at[pl.ds(j * _IW, _IW)],
                                deg_sh.at[rows_v.at[j]], add=True)
            return carry

        lax.fori_loop(0, _NCHUNK, chunk, 0)
        plsc.subcore_barrier()
        pltpu.sync_copy(deg_sh.at[pl.ds(sid * _SLICE, _SLICE)],
                        deg_hbm.at[pl.ds(cid * _NPAD + sid * _SLICE, _SLICE)])

    return pass1


def _make_pass2():
    mesh = plsc.VectorSubcoreMesh(core_axis_name="c", subcore_axis_name="s")

    @functools.partial(
        pl.kernel,
        mesh=mesh,
        compiler_params=pltpu.CompilerParams(needs_layout_passes=False),
        out_type=[jax.ShapeDtypeStruct((_E,), jnp.float32)],
        scratch_types=[
            pltpu.VMEM((_C,), jnp.int32),
            pltpu.VMEM((_C,), jnp.int32),
            pltpu.VMEM((_C,), jnp.float32),
            pltpu.VMEM((_C,), jnp.float32),
            pltpu.VMEM((_NPAD,), jnp.float32),
        ],
    )
    def pass2(dis_hbm, rows_hbm, cols_hbm, vraw_hbm,
              vout_hbm,
              rows_v, cols_v, vals_v, out_v, dis_full):
        cid = lax.axis_index("c")
        sid = lax.axis_index("s")
        wid = sid * _NC + cid
        pltpu.sync_copy(dis_hbm, dis_full)

        def chunk(c, carry):
            base = wid * _EPW + c * _C
            pltpu.sync_copy(rows_hbm.at[pl.ds(base, _C)], rows_v)
            pltpu.sync_copy(cols_hbm.at[pl.ds(base, _C)], cols_v)
            pltpu.sync_copy(vraw_hbm.at[pl.ds(base, _C)], vals_v)

            def vec(j, carry2):
                o = j * _L
                dr = plsc.load_gather(dis_full, [rows_v[pl.ds(o, _L)]])
                dc = plsc.load_gather(dis_full, [cols_v[pl.ds(o, _L)]])
                out_v[pl.ds(o, _L)] = vals_v[pl.ds(o, _L)] * dr * dc
                return carry2

            lax.fori_loop(0, _C // _L, vec, 0)
            pltpu.sync_copy(out_v, vout_hbm.at[pl.ds(base, _C)])
            return carry

        lax.fori_loop(0, _NCHUNK, chunk, 0)

    return pass2


_pass1_sr = _make_pass1(_RSR_PAD)
_pass1_tg = _make_pass1(_RTG_PAD)
_pass2 = _make_pass2()


def kernel(rel_sr_weight, rel_tg_weight, pos_sr, relation_sr, conf_sr,
           imp_sr, pca_sr, pos_tg, relation_tg, conf_tg, imp_tg, pca_tg):
    w_sr, w_tg = _relation_w(rel_sr_weight, rel_tg_weight)
    zeros = jnp.zeros((_NPAD,), jnp.float32)
    diag = jnp.arange(_N, dtype=jnp.int32)

    vraw_sr, deg_sr = _pass1_sr(w_sr, relation_sr, conf_sr, imp_sr, pca_sr,
                                pos_sr[0].reshape(_E // _IW, _IW), zeros)
    vraw_tg, deg_tg = _pass1_tg(w_tg, relation_tg, conf_tg, imp_tg, pca_tg,
                                pos_tg[0].reshape(_E // _IW, _IW), zeros)
    dis_sr, d2_sr, dis_tg, d2_tg = _deg_to_dis(deg_sr, deg_tg)

    out = []
    for pos, vraw, dis, d2 in ((pos_sr, vraw_sr, dis_sr, d2_sr),
                               (pos_tg, vraw_tg, dis_tg, d2_tg)):
        (vedge,) = _pass2(dis.reshape(-1), pos[0], pos[1], vraw)
        rows = jnp.concatenate([pos[0], diag])
        cols = jnp.concatenate([pos[1], diag])
        vals = jnp.concatenate([vedge, d2.reshape(-1)[:_N]])
        out.extend([rows, cols, vals])
    return tuple(out)


# speedup vs baseline: 419.6649x; 2.3639x over previous
import functools

import jax
import jax.numpy as jnp
from jax import lax
from jax.experimental import pallas as pl
from jax.experimental.pallas import tpu as pltpu
from jax.experimental.pallas import tpu_sc as plsc

_E = 3_200_000
_N = 100_000
_RSR_PAD = 1024
_RTG_PAD = 1280

_NC = 2
_NS = 16
_NW = _NC * _NS
_L = 16

_NPAD = 100_352
_SLICE = _NPAD // _NS
_EPW = _E // _NW
_C = 2000
_NCHUNK = _EPW // _C
_IW = 80
_IR = _C // _IW
_ERW = _EPW // _IW


def _relw_body(a_ref, b_ref, wsr_ref, wtg_ref):
    a = a_ref[...]
    b = b_ref[...]
    pa = a / (jnp.sqrt(jnp.sum(a * a, axis=1, keepdims=True)) + 1e-8)
    pb = b / (jnp.sqrt(jnp.sum(b * b, axis=1, keepdims=True)) + 1e-8)
    sim = lax.dot_general(pa, pb, (((1,), (1,)), ((), ())),
                          preferred_element_type=jnp.float32)
    ii = lax.broadcasted_iota(jnp.int32, sim.shape, 0)
    jj = lax.broadcasted_iota(jnp.int32, sim.shape, 1)
    neg = jnp.float32(-3.0e38)
    wsr_ref[...] = jnp.max(jnp.where(jj < 1200, sim, neg), axis=1, keepdims=True)
    wtg_ref[...] = jnp.max(jnp.where(ii < 1200, sim, neg), axis=0, keepdims=True)


def _relation_w(a, b):
    a = jnp.pad(a, ((0, 1280 - a.shape[0]), (0, 0)))
    b = jnp.pad(b, ((0, 1280 - b.shape[0]), (0, 0)))
    wsr2, wtg2 = pl.pallas_call(
        _relw_body,
        out_shape=(jax.ShapeDtypeStruct((1280, 1), jnp.float32),
                   jax.ShapeDtypeStruct((1, 1280), jnp.float32)),
    )(a, b)
    return wsr2[:_RSR_PAD, 0], wtg2[0, :]


def _dis_body(dsr_ref, dtg_ref, dis_sr_ref, d2_sr_ref, dis_tg_ref, d2_tg_ref):
    for dref, oref, o2ref in ((dsr_ref, dis_sr_ref, d2_sr_ref),
                              (dtg_ref, dis_tg_ref, d2_tg_ref)):
        d = dref[0] + dref[1] + 1.0
        y = jnp.where(d > 0.0, lax.rsqrt(jnp.maximum(d, 1e-12)), 0.0)
        oref[...] = y
        o2ref[...] = y * y


def _deg_to_dis(deg_sr, deg_tg):
    shp = jax.ShapeDtypeStruct((_NPAD // 128, 128), jnp.float32)
    return pl.pallas_call(
        _dis_body,
        out_shape=(shp, shp, shp, shp),
    )(deg_sr.reshape(2, _NPAD // 128, 128), deg_tg.reshape(2, _NPAD // 128, 128))


def _make_pass1(rpad):
    mesh = plsc.VectorSubcoreMesh(core_axis_name="c", subcore_axis_name="s")

    @functools.partial(
        pl.kernel,
        mesh=mesh,
        compiler_params=pltpu.CompilerParams(needs_layout_passes=False,
                                             use_tc_tiling_on_sc=False),
        out_type=[jax.ShapeDtypeStruct((_E,), jnp.float32),
                  jax.ShapeDtypeStruct((2 * _NPAD,), jnp.float32)],
        scratch_types=[
            pltpu.VMEM((rpad,), jnp.float32),
            [pltpu.VMEM((_C,), jnp.int32) for _ in range(2)],
            [pltpu.VMEM((_C,), jnp.float32) for _ in range(2)],
            [pltpu.VMEM((_C,), jnp.float32) for _ in range(2)],
            [pltpu.VMEM((_C,), jnp.float32) for _ in range(2)],
            [pltpu.VMEM((_IR, _IW), jnp.int32) for _ in range(2)],
            [pltpu.VMEM((_C,), jnp.float32) for _ in range(2)],
            pltpu.VMEM_SHARED((_NPAD,), jnp.float32),
            [pltpu.SemaphoreType.DMA for _ in range(2)],
            [pltpu.SemaphoreType.DMA for _ in range(2)],
            [pltpu.SemaphoreType.DMA for _ in range(2)],
            [pltpu.SemaphoreType.DMA for _ in range(2)],
        ],
    )
    def pass1(w_hbm, rel_hbm, conf_hbm, imp_hbm, pca_hbm, rows2_hbm, zeros_hbm,
              vals_hbm, deg_hbm,
              w_v, rel_v, conf_v, imp_v, pca_v, rows_v, vals_v, deg_sh,
              dsem, rsem, osem, ssem):
        cid = lax.axis_index("c")
        sid = lax.axis_index("s")
        wid = sid * _NC + cid
        pltpu.sync_copy(zeros_hbm.at[pl.ds(sid * _SLICE, _SLICE)],
                        deg_sh.at[pl.ds(sid * _SLICE, _SLICE)])
        pltpu.sync_copy(w_hbm, w_v)
        plsc.subcore_barrier()

        def stage4(c, b):
            base = wid * _EPW + c * _C
            return (pltpu.make_async_copy(rel_hbm.at[pl.ds(base, _C)], rel_v[b], dsem[b]),
                    pltpu.make_async_copy(conf_hbm.at[pl.ds(base, _C)], conf_v[b], dsem[b]),
                    pltpu.make_async_copy(imp_hbm.at[pl.ds(base, _C)], imp_v[b], dsem[b]),
                    pltpu.make_async_copy(pca_hbm.at[pl.ds(base, _C)], pca_v[b], dsem[b]))

        def rows_cp(c, b):
            rbase = wid * _ERW + c * _IR
            return pltpu.make_async_copy(rows2_hbm.at[pl.ds(rbase, _IR)],
                                         rows_v[b], rsem[b])

        def wb_cp(c, b):
            base = wid * _EPW + c * _C
            return pltpu.make_async_copy(vals_v[b],
                                         vals_hbm.at[pl.ds(base, _C)], osem[b])

        def scat_cps(b):
            return [pltpu.make_async_copy(vals_v[b].at[pl.ds(j * _IW, _IW)],
                                          deg_sh.at[rows_v[b].at[j]], ssem[b])
                    for j in range(_IR)]

        for b in range(2):
            for cp in stage4(b, b):
                cp.start()
            rows_cp(b, b).start()

        @pl.loop(0, _NCHUNK, step=2)
        def _(g):
            for b in range(2):
                c = g + b

                @pl.when(c >= 2)
                def _():
                    for cp in scat_cps(b):
                        cp.wait()
                    wb_cp(c - 2, b).wait()
                    rows_cp(c, b).start()

                for cp in stage4(c, b):
                    cp.wait()

                def vec(j, carry2):
                    o = j * _L
                    att = plsc.load_gather(w_v, [rel_v[b][pl.ds(o, _L)]])
                    v = (conf_v[b][pl.ds(o, _L)] * imp_v[b][pl.ds(o, _L)]
                         * (0.5 * pca_v[b][pl.ds(o, _L)] + 0.5 * att))
                    vals_v[b][pl.ds(o, _L)] = v
                    return carry2

                lax.fori_loop(0, _C // _L, vec, 0)
                rows_cp(c, b).wait()
                wb_cp(c, b).start()
                for cp in scat_cps(b):
                    cp.start(add=True)

                @pl.when(c + 2 < _NCHUNK)
                def _():
                    for cp in stage4(c + 2, b):
                        cp.start()

        for b in range(2):
            for cp in scat_cps(b):
                cp.wait()
            wb_cp(_NCHUNK - 2 + b, b).wait()
        plsc.subcore_barrier()
        pltpu.sync_copy(deg_sh.at[pl.ds(sid * _SLICE, _SLICE)],
                        deg_hbm.at[pl.ds(cid * _NPAD + sid * _SLICE, _SLICE)])

    return pass1


def _make_pass2():
    mesh = plsc.VectorSubcoreMesh(core_axis_name="c", subcore_axis_name="s")

    @functools.partial(
        pl.kernel,
        mesh=mesh,
        compiler_params=pltpu.CompilerParams(needs_layout_passes=False),
        out_type=[jax.ShapeDtypeStruct((_E,), jnp.float32)],
        scratch_types=[
            [pltpu.VMEM((_C,), jnp.int32) for _ in range(2)],
            [pltpu.VMEM((_C,), jnp.int32) for _ in range(2)],
            [pltpu.VMEM((_C,), jnp.float32) for _ in range(2)],
            [pltpu.VMEM((_C,), jnp.float32) for _ in range(2)],
            pltpu.VMEM((_NPAD,), jnp.float32),
            [pltpu.SemaphoreType.DMA for _ in range(2)],
            [pltpu.SemaphoreType.DMA for _ in range(2)],
        ],
    )
    def pass2(dis_hbm, rows_hbm, cols_hbm, vraw_hbm,
              vout_hbm,
              rows_v, cols_v, vals_v, out_v, dis_full, dsem, osem):
        cid = lax.axis_index("c")
        sid = lax.axis_index("s")
        wid = sid * _NC + cid
        pltpu.sync_copy(dis_hbm, dis_full)

        def stage3(c, b):
            base = wid * _EPW + c * _C
            return (pltpu.make_async_copy(rows_hbm.at[pl.ds(base, _C)], rows_v[b], dsem[b]),
                    pltpu.make_async_copy(cols_hbm.at[pl.ds(base, _C)], cols_v[b], dsem[b]),
                    pltpu.make_async_copy(vraw_hbm.at[pl.ds(base, _C)], vals_v[b], dsem[b]))

        def wb_cp(c, b):
            base = wid * _EPW + c * _C
            return pltpu.make_async_copy(out_v[b],
                                         vout_hbm.at[pl.ds(base, _C)], osem[b])

        for b in range(2):
            for cp in stage3(b, b):
                cp.start()

        @pl.loop(0, _NCHUNK, step=2)
        def _(g):
            for b in range(2):
                c = g + b

                @pl.when(c >= 2)
                def _():
                    wb_cp(c - 2, b).wait()

                for cp in stage3(c, b):
                    cp.wait()

                def vec(j, carry2):
                    o = j * _L
                    dr = plsc.load_gather(dis_full, [rows_v[b][pl.ds(o, _L)]])
                    dc = plsc.load_gather(dis_full, [cols_v[b][pl.ds(o, _L)]])
                    out_v[b][pl.ds(o, _L)] = vals_v[b][pl.ds(o, _L)] * dr * dc
                    return carry2

                lax.fori_loop(0, _C // _L, vec, 0)
                wb_cp(c, b).start()

                @pl.when(c + 2 < _NCHUNK)
                def _():
                    for cp in stage3(c + 2, b):
                        cp.start()

        for b in range(2):
            wb_cp(_NCHUNK - 2 + b, b).wait()

    return pass2


_pass1_sr = _make_pass1(_RSR_PAD)
_pass1_tg = _make_pass1(_RTG_PAD)
_pass2 = _make_pass2()


def kernel(rel_sr_weight, rel_tg_weight, pos_sr, relation_sr, conf_sr,
           imp_sr, pca_sr, pos_tg, relation_tg, conf_tg, imp_tg, pca_tg):
    w_sr, w_tg = _relation_w(rel_sr_weight, rel_tg_weight)
    zeros = jnp.zeros((_NPAD,), jnp.float32)
    diag = jnp.arange(_N, dtype=jnp.int32)

    vraw_sr, deg_sr = _pass1_sr(w_sr, relation_sr, conf_sr, imp_sr, pca_sr,
                                pos_sr[0].reshape(_E // _IW, _IW), zeros)
    vraw_tg, deg_tg = _pass1_tg(w_tg, relation_tg, conf_tg, imp_tg, pca_tg,
                                pos_tg[0].reshape(_E // _IW, _IW), zeros)
    dis_sr, d2_sr, dis_tg, d2_tg = _deg_to_dis(deg_sr, deg_tg)

    out = []
    for pos, vraw, dis, d2 in ((pos_sr, vraw_sr, dis_sr, d2_sr),
                               (pos_tg, vraw_tg, dis_tg, d2_tg)):
        (vedge,) = _pass2(dis.reshape(-1), pos[0], pos[1], vraw)
        rows = jnp.concatenate([pos[0], diag])
        cols = jnp.concatenate([pos[1], diag])
        vals = jnp.concatenate([vedge, d2.reshape(-1)[:_N]])
        out.extend([rows, cols, vals])
    return tuple(out)
